# baseline (speedup 1.0000x reference)
import functools

import jax
import jax.numpy as jnp
from jax.experimental import pallas as pl
from jax.experimental.pallas import tpu as pltpu
from jax.experimental.pallas import tpu_sc as plsc

_EPS = 1e-12


def _normalize_rows(x):
    norm = jnp.linalg.norm(x, axis=-1, keepdims=True)
    return x / jnp.maximum(norm, _EPS)


def _distance_argmin_kernel(zn_ref, en_ref, z2_ref, e2_ref,
                            idx_ref, enp_ref, acc_ref, *, n_tok, dim):
    i = pl.program_id(0)
    zn = zn_ref[...]
    en = en_ref[...]

    @pl.when(i == 0)
    def _():
        enp_ref[...] = jnp.concatenate([en, jnp.zeros_like(en)], axis=1)

    C = 2048
    K = e2_ref.shape[1]
    T = zn.shape[0]
    zb16 = zn.astype(jnp.bfloat16)
    z2 = z2_ref[...]
    accv = jnp.full((T,), jnp.inf, jnp.float32)
    accf = jnp.full((T,), jnp.inf, jnp.float32)
    acci = jnp.zeros((T,), jnp.int32)
    for c in range(K // C):
        dot = jax.lax.dot_general(
            zb16, en[c * C:(c + 1) * C].astype(jnp.bfloat16),
            (((1,), (1,)), ((), ())),
            preferred_element_type=jnp.float32)
        dist = z2 - 2.0 * dot + e2_ref[0, c * C:(c + 1) * C]
        lv = jnp.min(dist, axis=1)
        li = jnp.argmin(dist, axis=1).astype(jnp.int32) + c * C
        take = lv < accv
        accv = jnp.where(take, lv.astype(jnp.bfloat16).astype(jnp.float32),
                         accv)
        accf = jnp.where(take, lv, accf)
        acci = jnp.where(take, li, acci)
    idx_ref[0, 0, :] = acci
    dmin = accf

    @pl.when(i == 0)
    def _():
        acc_ref[...] = jnp.zeros_like(acc_ref)

    acc_ref[...] += jnp.sum(dmin)

    @pl.when(i == pl.num_programs(0) - 1)
    def _():
        acc_ref[...] *= 2.0 / (n_tok * dim)


def _sc_gather(table, indices2d):
    num_idx = indices2d.shape[1]
    dim = table.shape[1]
    window = 128
    mesh = plsc.VectorSubcoreMesh(
        core_axis_name="core", subcore_axis_name="subcore")

    @pl.kernel(out_type=jax.ShapeDtypeStruct((num_idx, dim), table.dtype),
               mesh=mesh)
    def gather_kernel(tab_hbm, i_hbm, o_hbm):
        def body(i_vmem, o_vmem):
            pltpu.sync_copy(tab_hbm.at[i_vmem.at[0]], o_vmem)

        pltpu.emit_pipeline(
            body,
            grid=(num_idx // window,),
            in_specs=[pl.BlockSpec((1, window), index_map=lambda i: (0, i))],
            out_specs=[pl.BlockSpec((window, dim),
                                    index_map=lambda i: (i, 0))],
            core_axis_name="subcore",
            dimension_semantics=(pltpu.PARALLEL,),
        )(i_hbm, o_hbm)

    return gather_kernel(table, indices2d)


def kernel(z, embedding):
    B, N, D = z.shape
    K = embedding.shape[0]
    n_tok = B * N
    T = 512
    n_blocks = n_tok // T
    z_flat = z.reshape(n_tok, D)

    zn = _normalize_rows(z_flat)
    en = _normalize_rows(embedding)
    z2 = jnp.sum(zn ** 2, axis=1, keepdims=True)
    e2 = jnp.sum(en ** 2, axis=1).reshape(1, K)

    idx3, enp, acc = pl.pallas_call(
        functools.partial(_distance_argmin_kernel, n_tok=n_tok, dim=D),
        grid=(n_blocks,),
        in_specs=[
            pl.BlockSpec((T, D), lambda i: (i, 0)),
            pl.BlockSpec((K, D), lambda i: (0, 0)),
            pl.BlockSpec((T, 1), lambda i: (i, 0)),
            pl.BlockSpec((1, K), lambda i: (0, 0)),
        ],
        out_specs=[
            pl.BlockSpec((1, 1, T), lambda i: (i, 0, 0)),
            pl.BlockSpec((K, 2 * D), lambda i: (0, 0)),
            pl.BlockSpec((1, 1), lambda i: (0, 0)),
        ],
        out_shape=[
            jax.ShapeDtypeStruct((n_blocks, 1, T), jnp.int32),
            jax.ShapeDtypeStruct((K, 2 * D), jnp.float32),
            jax.ShapeDtypeStruct((1, 1), jnp.float32),
        ],
    )(zn, en, z2, e2)

    indices = idx3.reshape(n_tok)
    quant = _sc_gather(enp, indices.reshape(1, n_tok))
    loss = acc.reshape(())
    return quant[:, :D].reshape(B, N, D), indices, loss


# speedup vs baseline: 1.1260x; 1.1260x over previous
import functools

import jax
import jax.numpy as jnp
from jax.experimental import pallas as pl
from jax.experimental.pallas import tpu as pltpu
from jax.experimental.pallas import tpu_sc as plsc

_EPS = 1e-12


def _normalize_rows(x):
    norm = jnp.linalg.norm(x, axis=-1, keepdims=True)
    return x / jnp.maximum(norm, _EPS)


def _distance_argmin_kernel(zn_ref, en_ref, z2_ref, e2_ref,
                            idx_ref, enp_ref, acc_ref, *, n_tok, dim):
    i = pl.program_id(0)
    zn = zn_ref[...]
    en = en_ref[...]

    kb = enp_ref.shape[0]
    ens = en_ref[pl.ds(i * kb, kb), :]
    enp_ref[...] = jnp.concatenate([ens, jnp.zeros_like(ens)], axis=1)

    C = 2048
    K = en.shape[0]
    T = zn.shape[0]
    zb16 = zn.astype(jnp.bfloat16)
    z2r = z2_ref[...]
    iota = jax.lax.broadcasted_iota(jnp.int32, (C, T), 0)
    accv = jnp.full((1, T), jnp.inf, jnp.float32)
    accf = jnp.full((1, T), jnp.inf, jnp.float32)
    acci = jnp.zeros((1, T), jnp.int32)
    for c in range(K // C):
        dot = jax.lax.dot_general(
            en[c * C:(c + 1) * C].astype(jnp.bfloat16), zb16,
            (((1,), (1,)), ((), ())),
            preferred_element_type=jnp.float32)
        dist = z2r - 2.0 * dot + e2_ref[c * C:(c + 1) * C]
        lv = jnp.min(dist, axis=0, keepdims=True)
        li = jnp.min(jnp.where(dist == lv, iota, K), axis=0,
                     keepdims=True).astype(jnp.int32) + c * C
        take = lv < accv
        accv = jnp.where(take, lv.astype(jnp.bfloat16).astype(jnp.float32),
                         accv)
        accf = jnp.where(take, lv, accf)
        acci = jnp.where(take, li, acci)
    idx_ref[0, 0, :] = acci[0, :]
    acc_ref[...] = jnp.sum(accf).reshape(1, 1, 1)


def _sc_gather(table, indices2d):
    num_idx = indices2d.shape[1]
    dim = table.shape[1]
    window = 128
    mesh = plsc.VectorSubcoreMesh(
        core_axis_name="core", subcore_axis_name="subcore")

    @pl.kernel(out_type=jax.ShapeDtypeStruct((num_idx, dim), table.dtype),
               mesh=mesh)
    def gather_kernel(tab_hbm, i_hbm, o_hbm):
        def body(i_vmem, o_vmem):
            pltpu.sync_copy(tab_hbm.at[i_vmem.at[0]], o_vmem)

        pltpu.emit_pipeline(
            body,
            grid=(num_idx // window,),
            in_specs=[pl.BlockSpec((1, window), index_map=lambda i: (0, i))],
            out_specs=[pl.BlockSpec((window, dim),
                                    index_map=lambda i: (i, 0))],
            core_axis_name="subcore",
            dimension_semantics=(pltpu.PARALLEL,),
        )(i_hbm, o_hbm)

    return gather_kernel(table, indices2d)


def kernel(z, embedding):
    B, N, D = z.shape
    K = embedding.shape[0]
    n_tok = B * N
    T = 1024
    n_blocks = n_tok // T
    z_flat = z.reshape(n_tok, D)

    zn = _normalize_rows(z_flat)
    en = _normalize_rows(embedding)
    z2 = jnp.sum(zn ** 2, axis=1).reshape(1, n_tok)
    e2 = jnp.sum(en ** 2, axis=1).reshape(K, 1)

    idx3, enp, acc = pl.pallas_call(
        functools.partial(_distance_argmin_kernel, n_tok=n_tok, dim=D),
        grid=(n_blocks,),
        in_specs=[
            pl.BlockSpec((T, D), lambda i: (i, 0)),
            pl.BlockSpec((K, D), lambda i: (0, 0)),
            pl.BlockSpec((1, T), lambda i: (0, i)),
            pl.BlockSpec((K, 1), lambda i: (0, 0)),
        ],
        out_specs=[
            pl.BlockSpec((1, 1, T), lambda i: (i, 0, 0)),
            pl.BlockSpec((K // n_blocks, 2 * D), lambda i: (i, 0)),
            pl.BlockSpec((1, 1, 1), lambda i: (i, 0, 0)),
        ],
        out_shape=[
            jax.ShapeDtypeStruct((n_blocks, 1, T), jnp.int32),
            jax.ShapeDtypeStruct((K, 2 * D), jnp.float32),
            jax.ShapeDtypeStruct((n_blocks, 1, 1), jnp.float32),
        ],
        compiler_params=pltpu.CompilerParams(
            dimension_semantics=("parallel",)),
    )(zn, en, z2, e2)

    indices = idx3.reshape(n_tok)
    quant = _sc_gather(enp, indices.reshape(1, n_tok))
    loss = jnp.sum(acc) * (2.0 / (n_tok * D))
    return quant[:, :D].reshape(B, N, D), indices, loss


# speedup vs baseline: 1.2975x; 1.1522x over previous
import functools

import jax
import jax.numpy as jnp
from jax.experimental import pallas as pl
from jax.experimental.pallas import tpu as pltpu
from jax.experimental.pallas import tpu_sc as plsc

_EPS = 1e-12


def _normalize_rows(x):
    norm = jnp.linalg.norm(x, axis=-1, keepdims=True)
    return x / jnp.maximum(norm, _EPS)


def _distance_argmin_kernel(zn_ref, en_ref, z2_ref, e2_ref,
                            idx_ref, enp_ref, acc_ref, *, n_tok, dim):
    i = pl.program_id(0)
    zn = zn_ref[...]
    en = en_ref[...]

    kb = enp_ref.shape[0]
    ens = en_ref[pl.ds(i * kb, kb), :]
    enp_ref[...] = jnp.concatenate([ens, jnp.zeros_like(ens)], axis=1)

    C = 2048
    K = en.shape[0]
    T = zn.shape[0]
    zb16 = zn.astype(jnp.bfloat16)
    z2r = z2_ref[...]
    accv = jnp.full((1, T), jnp.inf, jnp.float32)
    accf = jnp.full((1, T), jnp.inf, jnp.float32)
    acci = jnp.zeros((1, T), jnp.int32)
    for c in range(K // C):
        dot = jax.lax.dot_general(
            en[c * C:(c + 1) * C].astype(jnp.bfloat16), zb16,
            (((1,), (1,)), ((), ())),
            preferred_element_type=jnp.float32)
        dist = z2r - 2.0 * dot + e2_ref[c * C:(c + 1) * C]
        lv = jnp.min(dist, axis=0, keepdims=True)
        li = jnp.argmin(dist, axis=0).astype(jnp.int32).reshape(1, T) + c * C
        take = lv < accv
        accv = jnp.where(take, lv.astype(jnp.bfloat16).astype(jnp.float32),
                         accv)
        accf = jnp.where(take, lv, accf)
        acci = jnp.where(take, li, acci)
    idx_ref[0, 0, :] = acci[0, :]
    acc_ref[...] = jnp.sum(accf).reshape(1, 1, 1)


def _sc_gather(table, indices2d):
    num_idx = indices2d.shape[1]
    dim = table.shape[1]
    window = 128
    mesh = plsc.VectorSubcoreMesh(
        core_axis_name="core", subcore_axis_name="subcore")

    @pl.kernel(out_type=jax.ShapeDtypeStruct((num_idx, dim), table.dtype),
               mesh=mesh)
    def gather_kernel(tab_hbm, i_hbm, o_hbm):
        def body(i_vmem, o_vmem):
            pltpu.sync_copy(tab_hbm.at[i_vmem.at[0]], o_vmem)

        pltpu.emit_pipeline(
            body,
            grid=(num_idx // window,),
            in_specs=[pl.BlockSpec((1, window), index_map=lambda i: (0, i))],
            out_specs=[pl.BlockSpec((window, dim),
                                    index_map=lambda i: (i, 0))],
            core_axis_name="subcore",
            dimension_semantics=(pltpu.PARALLEL,),
        )(i_hbm, o_hbm)

    return gather_kernel(table, indices2d)


def kernel(z, embedding):
    B, N, D = z.shape
    K = embedding.shape[0]
    n_tok = B * N
    T = 1024
    n_blocks = n_tok // T
    z_flat = z.reshape(n_tok, D)

    zn = _normalize_rows(z_flat)
    en = _normalize_rows(embedding)
    z2 = jnp.sum(zn ** 2, axis=1).reshape(1, n_tok)
    e2 = jnp.sum(en ** 2, axis=1).reshape(K, 1)

    idx3, enp, acc = pl.pallas_call(
        functools.partial(_distance_argmin_kernel, n_tok=n_tok, dim=D),
        grid=(n_blocks,),
        in_specs=[
            pl.BlockSpec((T, D), lambda i: (i, 0)),
            pl.BlockSpec((K, D), lambda i: (0, 0)),
            pl.BlockSpec((1, T), lambda i: (0, i)),
            pl.BlockSpec((K, 1), lambda i: (0, 0)),
        ],
        out_specs=[
            pl.BlockSpec((1, 1, T), lambda i: (i, 0, 0)),
            pl.BlockSpec((K // n_blocks, 2 * D), lambda i: (i, 0)),
            pl.BlockSpec((1, 1, 1), lambda i: (i, 0, 0)),
        ],
        out_shape=[
            jax.ShapeDtypeStruct((n_blocks, 1, T), jnp.int32),
            jax.ShapeDtypeStruct((K, 2 * D), jnp.float32),
            jax.ShapeDtypeStruct((n_blocks, 1, 1), jnp.float32),
        ],
        compiler_params=pltpu.CompilerParams(
            dimension_semantics=("parallel",)),
    )(zn, en, z2, e2)

    indices = idx3.reshape(n_tok)
    quant = _sc_gather(enp, indices.reshape(1, n_tok))
    loss = jnp.sum(acc) * (2.0 / (n_tok * D))
    return quant[:, :D].reshape(B, N, D), indices, loss
